# baseline (speedup 1.0000x reference)
import functools

import jax
import jax.numpy as jnp
from jax import lax
from jax.experimental import pallas as pl
from jax.experimental.pallas import tpu as pltpu
from jax.experimental.pallas import tpu_sc as plsc

NUM_TOOLS = 256
D_TOOL = 64
D_PAD = 128
BATCH = 16384

_NUM_CORES = 2
_NUM_SUBCORES = 16
_NW = _NUM_CORES * _NUM_SUBCORES
_BPW = BATCH // _NW
_CHUNK = 128
_NCHUNK = _BPW // _CHUNK

_mesh = plsc.VectorSubcoreMesh(core_axis_name="c", subcore_axis_name="s")


@functools.partial(
    pl.kernel,
    mesh=_mesh,
    out_type=jax.ShapeDtypeStruct((BATCH, D_PAD), jnp.float32),
    scratch_types=[
        pltpu.VMEM((_BPW,), jnp.int32),
        pltpu.VMEM((_BPW, D_PAD), jnp.float32),
        pltpu.SemaphoreType.DMA((_NCHUNK,)),
        pltpu.SemaphoreType.DMA,
    ],
    compiler_params=pltpu.CompilerParams(
        disable_bounds_checks=True,
        disable_semaphore_checks=True,
    ),
)
def _gather_kernel(idx_hbm, table_hbm, out_hbm, idx_v, rows_v, gsem, ssem):
    wid = lax.axis_index("s") * _NUM_CORES + lax.axis_index("c")
    base = wid * _BPW
    pltpu.sync_copy(idx_hbm.at[pl.ds(base, _BPW)], idx_v)
    gathers = [
        pltpu.async_copy(
            table_hbm.at[idx_v.at[pl.ds(j * _CHUNK, _CHUNK)]],
            rows_v.at[pl.ds(j * _CHUNK, _CHUNK)],
            gsem.at[j],
        )
        for j in range(_NCHUNK)
    ]
    stores = []
    for j in range(_NCHUNK):
        gathers[j].wait()
        stores.append(
            pltpu.async_copy(
                rows_v.at[pl.ds(j * _CHUNK, _CHUNK)],
                out_hbm.at[pl.ds(base + j * _CHUNK, _CHUNK)],
                ssem,
            )
        )
    for s in stores:
        s.wait()


def kernel(indices, table):
    table_pad = jnp.pad(table, ((0, 0), (0, D_PAD - D_TOOL)))
    out_pad = _gather_kernel(indices, table_pad)
    return out_pad[:, :D_TOOL]


# speedup vs baseline: 1.1649x; 1.1649x over previous
import functools

import jax
import jax.numpy as jnp
from jax import lax
from jax.experimental import pallas as pl
from jax.experimental.pallas import tpu as pltpu
from jax.experimental.pallas import tpu_sc as plsc

NUM_TOOLS = 256
D_TOOL = 64
D_PAD = 128
BATCH = 16384

_NUM_CORES = 2
_NUM_SUBCORES = 16
_NW = _NUM_CORES * _NUM_SUBCORES
_BPW = BATCH // _NW
_CHUNK = 128
_NCHUNK = _BPW // _CHUNK

_mesh = plsc.VectorSubcoreMesh(core_axis_name="c", subcore_axis_name="s")


@functools.partial(
    pl.kernel,
    mesh=_mesh,
    out_type=jax.ShapeDtypeStruct((BATCH, D_PAD), jnp.float32),
    scratch_types=[
        pltpu.VMEM((_BPW,), jnp.int32),
        pltpu.VMEM((_BPW, D_TOOL), jnp.float32),
        pltpu.SemaphoreType.DMA((_NCHUNK,)),
        pltpu.SemaphoreType.DMA,
    ],
    compiler_params=pltpu.CompilerParams(
        use_tc_tiling_on_sc=False,
        disable_bounds_checks=True,
        disable_semaphore_checks=True,
    ),
)
def _gather_kernel(idx_hbm, table_hbm, out_hbm, idx_v, rows_v, gsem, ssem):
    wid = lax.axis_index("s") * _NUM_CORES + lax.axis_index("c")
    base = wid * _BPW
    pltpu.sync_copy(idx_hbm.at[pl.ds(base, _BPW)], idx_v)
    gathers = [
        pltpu.async_copy(
            table_hbm.at[idx_v.at[pl.ds(j * _CHUNK, _CHUNK)]],
            rows_v.at[pl.ds(j * _CHUNK, _CHUNK)],
            gsem.at[j],
        )
        for j in range(_NCHUNK)
    ]
    stores = []
    for j in range(_NCHUNK):
        gathers[j].wait()
        stores.append(
            pltpu.async_copy(
                rows_v.at[pl.ds(j * _CHUNK, _CHUNK)],
                out_hbm.at[pl.ds(base + j * _CHUNK, _CHUNK), pl.ds(0, D_TOOL)],
                ssem,
            )
        )
    for s in stores:
        s.wait()


def kernel(indices, table):
    out_pad = _gather_kernel(indices, table)
    return out_pad[:, :D_TOOL]


# speedup vs baseline: 1.1973x; 1.0278x over previous
import functools

import jax
import jax.numpy as jnp
from jax import lax
from jax.experimental import pallas as pl
from jax.experimental.pallas import tpu as pltpu
from jax.experimental.pallas import tpu_sc as plsc

NUM_TOOLS = 256
D_TOOL = 64
D_PAD = 128
BATCH = 16384

_NUM_CORES = 2
_NUM_SUBCORES = 16
_NW = _NUM_CORES * _NUM_SUBCORES
_BPW = BATCH // _NW
_CHUNK = 512
_NCHUNK = _BPW // _CHUNK

_mesh = plsc.VectorSubcoreMesh(core_axis_name="c", subcore_axis_name="s")


@functools.partial(
    pl.kernel,
    mesh=_mesh,
    out_type=jax.ShapeDtypeStruct((BATCH, D_PAD), jnp.float32),
    scratch_types=[
        pltpu.VMEM((_BPW,), jnp.int32),
        pltpu.VMEM((_BPW, D_TOOL), jnp.float32),
        pltpu.SemaphoreType.DMA((_NCHUNK,)),
        pltpu.SemaphoreType.DMA,
    ],
    compiler_params=pltpu.CompilerParams(
        use_tc_tiling_on_sc=False,
        disable_bounds_checks=True,
        disable_semaphore_checks=True,
    ),
)
def _gather_kernel(idx_hbm, table_hbm, out_hbm, idx_v, rows_v, gsem, ssem):
    wid = lax.axis_index("s") * _NUM_CORES + lax.axis_index("c")
    base = wid * _BPW
    pltpu.sync_copy(idx_hbm.at[pl.ds(base, _BPW)], idx_v)
    gathers = [
        pltpu.async_copy(
            table_hbm.at[idx_v.at[pl.ds(j * _CHUNK, _CHUNK)]],
            rows_v.at[pl.ds(j * _CHUNK, _CHUNK)],
            gsem.at[j],
        )
        for j in range(_NCHUNK)
    ]
    stores = []
    for j in range(_NCHUNK):
        gathers[j].wait()
        stores.append(
            pltpu.async_copy(
                rows_v.at[pl.ds(j * _CHUNK, _CHUNK)],
                out_hbm.at[pl.ds(base + j * _CHUNK, _CHUNK), pl.ds(0, D_TOOL)],
                ssem,
            )
        )
    for s in stores:
        s.wait()


def kernel(indices, table):
    out_pad = _gather_kernel(indices, table)
    return out_pad[:, :D_TOOL]
